# Initial kernel scaffold; baseline (speedup 1.0000x reference)
import jax
import jax.numpy as jnp
from jax.experimental import pallas as pl


def kernel(x, edge_index, W1, b1, W2, b2, Wp, bp):
    raise NotImplementedError("write your pallas kernel here")


# speedup vs baseline: 8.5812x; 8.5812x over previous
import functools

import jax
import jax.numpy as jnp
from jax import lax
from jax.experimental import pallas as pl
from jax.experimental.pallas import tpu as pltpu
from jax.experimental.pallas import tpu_sc as plsc

N_NODES = 10000
N_EDGES = 320000
D = 128

NC = 2
NS = 16
NW = NC * NS

NPAD = 10240
CHUNK = 128
CHUNKS = 80
EPAD = NW * CHUNKS * CHUNK
ROWS_PER_TILE = NPAD // NS
HW = 16

BLK = 1024
GRID = NPAD // BLK

_mesh = plsc.VectorSubcoreMesh(core_axis_name="c", subcore_axis_name="s")


def _worker_id():
    return lax.axis_index("s") * NC + lax.axis_index("c")


@functools.partial(
    pl.kernel,
    out_type=jax.ShapeDtypeStruct((NC, NPAD, HW), jnp.float32),
    mesh=_mesh,
    scratch_types=[
        pltpu.VMEM((CHUNKS, CHUNK), jnp.int32),
        pltpu.VMEM((CHUNK, HW), jnp.float32),
        pltpu.VMEM((CHUNK, HW), jnp.float32),
        pltpu.VMEM_SHARED((NPAD, HW), jnp.float32),
        pltpu.SemaphoreType.DMA,
    ],
)
def _sc_degree(dst_hbm, hist_hbm, dst_v, ones_v, zero_v, acc_sh, sem):
    c = lax.axis_index("c")
    s = lax.axis_index("s")
    wid = _worker_id()

    @pl.loop(0, CHUNK)
    def _(j):
        ones_v[j, :] = jnp.ones((HW,), jnp.float32)
        zero_v[j, :] = jnp.zeros((HW,), jnp.float32)

    for t in range(ROWS_PER_TILE // CHUNK):
        pltpu.sync_copy(zero_v, acc_sh.at[pl.ds(s * ROWS_PER_TILE + t * CHUNK, CHUNK)])
    pltpu.sync_copy(dst_hbm.at[pl.ds(wid * CHUNKS, CHUNKS)], dst_v)
    plsc.subcore_barrier()

    @pl.loop(0, CHUNKS)
    def _(j):
        pltpu.sync_copy(ones_v, acc_sh.at[dst_v.at[j]], add=True)

    plsc.subcore_barrier()
    base = s * ROWS_PER_TILE
    for t in range(ROWS_PER_TILE // CHUNK):
        pltpu.sync_copy(acc_sh.at[pl.ds(base + t * CHUNK, CHUNK)], zero_v)
        pltpu.sync_copy(zero_v, hist_hbm.at[c, pl.ds(base + t * CHUNK, CHUNK)])


@functools.partial(
    pl.kernel,
    out_type=jax.ShapeDtypeStruct((NC, NPAD, D), jnp.float32),
    mesh=_mesh,
    scratch_types=[
        pltpu.VMEM((CHUNKS, CHUNK), jnp.int32),
        pltpu.VMEM((CHUNKS, CHUNK), jnp.int32),
        pltpu.VMEM((CHUNK, D), jnp.float32),
        pltpu.VMEM_SHARED((NPAD, D), jnp.float32),
        pltpu.SemaphoreType.DMA,
    ],
)
def _sc_scatter(g_hbm, src_hbm, dst_hbm, acc_hbm, src_v, dst_v, rows_v, acc_sh, sem):
    c = lax.axis_index("c")
    s = lax.axis_index("s")
    wid = _worker_id()

    @pl.loop(0, CHUNK)
    def _(j):
        for k in range(D // 16):
            rows_v[j, pl.ds(k * 16, 16)] = jnp.zeros((16,), jnp.float32)

    for t in range(ROWS_PER_TILE // CHUNK):
        pltpu.sync_copy(rows_v, acc_sh.at[pl.ds(s * ROWS_PER_TILE + t * CHUNK, CHUNK)])
    pltpu.sync_copy(src_hbm.at[pl.ds(wid * CHUNKS, CHUNKS)], src_v)
    pltpu.sync_copy(dst_hbm.at[pl.ds(wid * CHUNKS, CHUNKS)], dst_v)
    plsc.subcore_barrier()

    @pl.loop(0, CHUNKS)
    def _(j):
        pltpu.async_copy(g_hbm.at[src_v.at[j]], rows_v, sem).wait()
        pltpu.sync_copy(rows_v, acc_sh.at[dst_v.at[j]], add=True)

    plsc.subcore_barrier()
    base = s * ROWS_PER_TILE
    for t in range(ROWS_PER_TILE // CHUNK):
        pltpu.sync_copy(acc_sh.at[pl.ds(base + t * CHUNK, CHUNK)], rows_v)
        pltpu.sync_copy(rows_v, acc_hbm.at[c, pl.ds(base + t * CHUNK, CHUNK)])


def _dinv(hist_ref):
    deg = hist_ref[0, :, 0:1] + hist_ref[1, :, 0:1] + 1.0
    return lax.rsqrt(deg)


def _tc_g1_body(x_ref, w_ref, hist_ref, g_ref):
    h = jnp.dot(x_ref[...], w_ref[...], preferred_element_type=jnp.float32,
                precision=lax.Precision.HIGHEST)
    g_ref[...] = h * _dinv(hist_ref)


def _tc_mid_body(acc_ref, g_ref, hist_ref, b_ref, w_ref, out_ref):
    dinv = _dinv(hist_ref)
    z = dinv * (acc_ref[0] + acc_ref[1] + g_ref[...]) + b_ref[...]
    z = jnp.maximum(z, 0.0)
    h = jnp.dot(z, w_ref[...], preferred_element_type=jnp.float32,
                precision=lax.Precision.HIGHEST)
    out_ref[...] = h * dinv


def _tc_final_body(acc_ref, g_ref, hist_ref, b_ref, w_ref, bp_ref, out_ref):
    dinv = _dinv(hist_ref)
    z = dinv * (acc_ref[0] + acc_ref[1] + g_ref[...]) + b_ref[...]
    out_ref[...] = jnp.dot(z, w_ref[...], preferred_element_type=jnp.float32,
                           precision=lax.Precision.HIGHEST) + bp_ref[...]


_row_spec = pl.BlockSpec((BLK, D), lambda i: (i, 0))
_acc_spec = pl.BlockSpec((NC, BLK, D), lambda i: (0, i, 0))
_hist_spec = pl.BlockSpec((NC, BLK, HW), lambda i: (0, i, 0))
_full_spec = pl.BlockSpec((D, D), lambda i: (0, 0))
_bias_spec = pl.BlockSpec((1, D), lambda i: (0, 0))
_out_struct = jax.ShapeDtypeStruct((NPAD, D), jnp.float32)


def kernel(x, edge_index, W1, b1, W2, b2, Wp, bp):
    src = edge_index[0]
    dst = edge_index[1]
    pad = jnp.full((EPAD - N_EDGES,), NPAD - 1, dtype=jnp.int32)
    srcR = jnp.concatenate([src, pad]).reshape(NW * CHUNKS, CHUNK)
    dstR = jnp.concatenate([dst, pad]).reshape(NW * CHUNKS, CHUNK)
    x_p = jnp.zeros((NPAD, D), jnp.float32).at[:N_NODES].set(x)
    b1r = b1.reshape(1, D)
    b2r = b2.reshape(1, D)
    bpr = bp.reshape(1, D)

    hist = _sc_degree(dstR)

    g1 = pl.pallas_call(
        _tc_g1_body,
        grid=(GRID,),
        in_specs=[_row_spec, _full_spec, _hist_spec],
        out_specs=_row_spec,
        out_shape=_out_struct,
    )(x_p, W1, hist)

    acc1 = _sc_scatter(g1, srcR, dstR)

    g2 = pl.pallas_call(
        _tc_mid_body,
        grid=(GRID,),
        in_specs=[_acc_spec, _row_spec, _hist_spec, _bias_spec, _full_spec],
        out_specs=_row_spec,
        out_shape=_out_struct,
    )(acc1, g1, hist, b1r, W2)

    acc2 = _sc_scatter(g2, srcR, dstR)

    out = pl.pallas_call(
        _tc_final_body,
        grid=(GRID,),
        in_specs=[_acc_spec, _row_spec, _hist_spec, _bias_spec, _full_spec,
                  _bias_spec],
        out_specs=_row_spec,
        out_shape=_out_struct,
    )(acc2, g2, hist, b2r, Wp, bpr)

    return out[:N_NODES]


# speedup vs baseline: 9.4117x; 1.0968x over previous
import functools

import jax
import jax.numpy as jnp
from jax import lax
from jax.experimental import pallas as pl
from jax.experimental.pallas import tpu as pltpu
from jax.experimental.pallas import tpu_sc as plsc

N_NODES = 10000
N_EDGES = 320000
D = 128

NC = 2
NS = 16
NW = NC * NS

NPAD = 10240
CHUNK = 64
CHUNKS = 160
DCHUNK = 128
DCHUNKS = 80
EPAD = NW * CHUNKS * CHUNK
ROWS_PER_TILE = NPAD // NS
HW = 16

BLK = 1024
GRID = NPAD // BLK

_mesh = plsc.VectorSubcoreMesh(core_axis_name="c", subcore_axis_name="s")


def _worker_id():
    return lax.axis_index("s") * NC + lax.axis_index("c")


@functools.partial(
    pl.kernel,
    out_type=jax.ShapeDtypeStruct((NC, NPAD, HW), jnp.float32),
    mesh=_mesh,
    scratch_types=[
        pltpu.VMEM((DCHUNKS, DCHUNK), jnp.int32),
        pltpu.VMEM((DCHUNK, HW), jnp.float32),
        pltpu.VMEM((DCHUNK, HW), jnp.float32),
        pltpu.VMEM_SHARED((NPAD, HW), jnp.float32),
        pltpu.SemaphoreType.DMA,
    ],
)
def _sc_degree(dst_hbm, hist_hbm, dst_v, ones_v, zero_v, acc_sh, sem):
    c = lax.axis_index("c")
    s = lax.axis_index("s")
    wid = _worker_id()

    @pl.loop(0, DCHUNK)
    def _(j):
        ones_v[j, :] = jnp.ones((HW,), jnp.float32)
        zero_v[j, :] = jnp.zeros((HW,), jnp.float32)

    for t in range(ROWS_PER_TILE // DCHUNK):
        pltpu.sync_copy(zero_v, acc_sh.at[pl.ds(s * ROWS_PER_TILE + t * DCHUNK, DCHUNK)])
    pltpu.sync_copy(dst_hbm.at[pl.ds(wid * DCHUNKS, DCHUNKS)], dst_v)
    plsc.subcore_barrier()

    @pl.loop(0, DCHUNKS)
    def _(j):
        pltpu.sync_copy(ones_v, acc_sh.at[dst_v.at[j]], add=True)

    plsc.subcore_barrier()
    base = s * ROWS_PER_TILE
    for t in range(ROWS_PER_TILE // DCHUNK):
        pltpu.sync_copy(acc_sh.at[pl.ds(base + t * DCHUNK, DCHUNK)], zero_v)
        pltpu.sync_copy(zero_v, hist_hbm.at[c, pl.ds(base + t * DCHUNK, DCHUNK)])


NBUF = 2
IBLK = 32
NSB = CHUNKS // IBLK


@functools.partial(
    pl.kernel,
    out_type=jax.ShapeDtypeStruct((NC, NPAD, D), jnp.float32),
    mesh=_mesh,
    scratch_types=[
        pltpu.VMEM((IBLK, CHUNK), jnp.int32),
        pltpu.VMEM((IBLK, CHUNK), jnp.int32),
        [pltpu.VMEM((CHUNK, D), jnp.float32) for _ in range(NBUF)],
        pltpu.VMEM_SHARED((NPAD, D), jnp.float32),
        [pltpu.SemaphoreType.DMA for _ in range(NBUF)],
        [pltpu.SemaphoreType.DMA for _ in range(NBUF)],
    ],
)
def _sc_scatter(g_hbm, src_hbm, dst_hbm, acc_hbm, src_v, dst_v, rows, acc_sh,
                gsem, ssem):
    c = lax.axis_index("c")
    s = lax.axis_index("s")
    wid = _worker_id()

    @pl.loop(0, CHUNK)
    def _(j):
        for k in range(D // 16):
            rows[0][j, pl.ds(k * 16, 16)] = jnp.zeros((16,), jnp.float32)

    for t in range(ROWS_PER_TILE // CHUNK):
        pltpu.sync_copy(rows[0], acc_sh.at[pl.ds(s * ROWS_PER_TILE + t * CHUNK, CHUNK)])
    plsc.subcore_barrier()

    for sb in range(NSB):
        rbase = wid * CHUNKS + sb * IBLK
        pltpu.sync_copy(src_hbm.at[pl.ds(rbase, IBLK)], src_v)
        pltpu.sync_copy(dst_hbm.at[pl.ds(rbase, IBLK)], dst_v)
        for b in range(NBUF):
            pltpu.async_copy(g_hbm.at[src_v.at[b]], rows[b], gsem[b])

        @pl.loop(0, IBLK - NBUF, step=NBUF)
        def _(j):
            for b in range(NBUF):
                pltpu.make_async_copy(g_hbm.at[src_v.at[j + b]], rows[b],
                                      gsem[b]).wait()
                pltpu.sync_copy(rows[b], acc_sh.at[dst_v.at[j + b]], add=True)
                pltpu.async_copy(g_hbm.at[src_v.at[j + b + NBUF]], rows[b],
                                 gsem[b])

        for b in range(NBUF):
            jj = IBLK - NBUF + b
            pltpu.make_async_copy(g_hbm.at[src_v.at[jj]], rows[b],
                                  gsem[b]).wait()
            pltpu.sync_copy(rows[b], acc_sh.at[dst_v.at[jj]], add=True)

    plsc.subcore_barrier()
    base = s * ROWS_PER_TILE
    for t in range(ROWS_PER_TILE // CHUNK):
        pltpu.sync_copy(acc_sh.at[pl.ds(base + t * CHUNK, CHUNK)], rows[0])
        pltpu.sync_copy(rows[0], acc_hbm.at[c, pl.ds(base + t * CHUNK, CHUNK)])


def _dinv(hist_ref):
    deg = hist_ref[0, :, 0:1] + hist_ref[1, :, 0:1] + 1.0
    return lax.rsqrt(deg)


def _tc_g1_body(x_ref, w_ref, hist_ref, g_ref):
    h = jnp.dot(x_ref[...], w_ref[...], preferred_element_type=jnp.float32,
                precision=lax.Precision.HIGHEST)
    g_ref[...] = h * _dinv(hist_ref)


def _tc_mid_body(acc_ref, g_ref, hist_ref, b_ref, w_ref, out_ref):
    dinv = _dinv(hist_ref)
    z = dinv * (acc_ref[0] + acc_ref[1] + g_ref[...]) + b_ref[...]
    z = jnp.maximum(z, 0.0)
    h = jnp.dot(z, w_ref[...], preferred_element_type=jnp.float32,
                precision=lax.Precision.HIGHEST)
    out_ref[...] = h * dinv


def _tc_final_body(acc_ref, g_ref, hist_ref, b_ref, w_ref, bp_ref, out_ref):
    dinv = _dinv(hist_ref)
    z = dinv * (acc_ref[0] + acc_ref[1] + g_ref[...]) + b_ref[...]
    out_ref[...] = jnp.dot(z, w_ref[...], preferred_element_type=jnp.float32,
                           precision=lax.Precision.HIGHEST) + bp_ref[...]


_row_spec = pl.BlockSpec((BLK, D), lambda i: (i, 0))
_acc_spec = pl.BlockSpec((NC, BLK, D), lambda i: (0, i, 0))
_hist_spec = pl.BlockSpec((NC, BLK, HW), lambda i: (0, i, 0))
_full_spec = pl.BlockSpec((D, D), lambda i: (0, 0))
_bias_spec = pl.BlockSpec((1, D), lambda i: (0, 0))
_out_struct = jax.ShapeDtypeStruct((NPAD, D), jnp.float32)


def kernel(x, edge_index, W1, b1, W2, b2, Wp, bp):
    src = edge_index[0]
    dst = edge_index[1]
    pad = jnp.full((EPAD - N_EDGES,), NPAD - 1, dtype=jnp.int32)
    src_p = jnp.concatenate([src, pad])
    dst_p = jnp.concatenate([dst, pad])
    srcR = src_p.reshape(NW * CHUNKS, CHUNK)
    dstR = dst_p.reshape(NW * CHUNKS, CHUNK)
    dstD = dst_p.reshape(NW * DCHUNKS, DCHUNK)
    x_p = jnp.zeros((NPAD, D), jnp.float32).at[:N_NODES].set(x)
    b1r = b1.reshape(1, D)
    b2r = b2.reshape(1, D)
    bpr = bp.reshape(1, D)

    hist = _sc_degree(dstD)

    g1 = pl.pallas_call(
        _tc_g1_body,
        grid=(GRID,),
        in_specs=[_row_spec, _full_spec, _hist_spec],
        out_specs=_row_spec,
        out_shape=_out_struct,
    )(x_p, W1, hist)

    acc1 = _sc_scatter(g1, srcR, dstR)

    g2 = pl.pallas_call(
        _tc_mid_body,
        grid=(GRID,),
        in_specs=[_acc_spec, _row_spec, _hist_spec, _bias_spec, _full_spec],
        out_specs=_row_spec,
        out_shape=_out_struct,
    )(acc1, g1, hist, b1r, W2)

    acc2 = _sc_scatter(g2, srcR, dstR)

    out = pl.pallas_call(
        _tc_final_body,
        grid=(GRID,),
        in_specs=[_acc_spec, _row_spec, _hist_spec, _bias_spec, _full_spec,
                  _bias_spec],
        out_specs=_row_spec,
        out_shape=_out_struct,
    )(acc2, g2, hist, b2r, Wp, bpr)

    return out[:N_NODES]
